# baseline (speedup 1.0000x reference)
import functools

import jax
import jax.numpy as jnp
from jax import lax
from jax.experimental import pallas as pl
from jax.experimental.pallas import tpu as pltpu
from jax.experimental.pallas import tpu_sc as plsc

N = 1600000
C = 50000
NW = 32
NS = 64
CPV = 782
CPW = 2 * CPV
RPAD = 1568
OUTR = NW * CPW
TP = 640
GPT = TP // 16

BLKN = 6400

WG1O, BG1O, WG2O, BG2O = 0, 1024, 1056, 2080
WTOT = 2112


def _mlp_body(p_ref, w1_ref, b1_ref, w2_ref, b2_ref, h_ref):
    p = p_ref[...]
    h1 = jnp.maximum(
        jnp.dot(w1_ref[...], p, preferred_element_type=jnp.float32)
        + b1_ref[...], 0.0)
    h2 = jnp.maximum(
        jnp.dot(w2_ref[...], h1, preferred_element_type=jnp.float32)
        + b2_ref[...], 0.0)
    h_ref[...] = h2


_point_mlp = pl.pallas_call(
    _mlp_body,
    grid=(N // BLKN,),
    in_specs=[
        pl.BlockSpec((3, BLKN), lambda i: (0, i)),
        pl.BlockSpec((16, 3), lambda i: (0, 0)),
        pl.BlockSpec((16, 1), lambda i: (0, 0)),
        pl.BlockSpec((32, 16), lambda i: (0, 0)),
        pl.BlockSpec((32, 1), lambda i: (0, 0)),
    ],
    out_specs=pl.BlockSpec((32, BLKN), lambda i: (0, i)),
    out_shape=jax.ShapeDtypeStruct((32, N), jnp.float32),
)


def _splat(s):
    return jnp.full((16,), s, dtype=jnp.float32)


def _pool_body(ht_h, cl_h, w_h, bd_h, out_h,
               htbA, htbB, clbA, clbB, wb, bdb, g1b, poolb, bsm):
    wid = lax.axis_index("s") * 2 + lax.axis_index("c")
    pltpu.sync_copy(w_h, wb)
    pltpu.sync_copy(bd_h, bdb)
    iota = lax.iota(jnp.int32, 16)
    iota32 = iota * 32
    rows1 = iota + 16
    c_loW = wid * CPW
    c_loA = c_loW
    c_loB = c_loW + CPV

    for i in range(5):
        v = bdb[pl.ds(i * 16, 16)]
        for l in range(16):
            bsm[i * 16 + l] = v[l]

    zf = jnp.zeros((16,), jnp.float32)

    def zbody(i, carry):
        poolb[pl.ds(pl.multiple_of(i * 16, 16), 16)] = zf
        return carry

    lax.fori_loop(0, RPAD * 32 // 16, zbody, 0)

    startA = bsm[2 * wid]
    endA = bsm[2 * wid + 1]
    startB = bsm[2 * wid + 1]
    endB = bsm[2 * wid + 2]
    t0A = startA // TP
    t1A = (endA + TP - 1) // TP
    t0B = startB // TP
    t1B = (endB + TP - 1) // TP
    nt = jnp.maximum(t1A - t0A, t1B - t0B)

    def tile_body(t, carry):
        tA = t0A + t
        tB = t0B + t
        vA = tA < t1A
        vB = tB < t1B
        tbA = tA * TP
        tbB = tB * TP

        @pl.when(vA)
        def _():
            off = pl.multiple_of(tbA, TP)
            pltpu.sync_copy(ht_h.at[:, pl.ds(off, TP)], htbA)
            pltpu.sync_copy(cl_h.at[pl.ds(off, TP)], clbA)

        @pl.when(vB)
        def _():
            off = pl.multiple_of(tbB, TP)
            pltpu.sync_copy(ht_h.at[:, pl.ds(off, TP)], htbB)
            pltpu.sync_copy(cl_h.at[pl.ds(off, TP)], clbB)

        g_loA = jnp.where(vA, (jnp.maximum(startA, tbA) - tbA) // 16, GPT)
        g_hiA = jnp.where(vA, (jnp.minimum(endA, tbA + TP) - tbA + 15) // 16,
                          0)
        g_loB = jnp.where(vB, (jnp.maximum(startB, tbB) - tbB) // 16, GPT)
        g_hiB = jnp.where(vB, (jnp.minimum(endB, tbB + TP) - tbB + 15) // 16,
                          0)

        def grp(g, carry):
            prevA, a0A, a1A, prevB, a0B, a1B = carry
            gvA = jnp.logical_and(g >= g_loA, g < g_hiA)
            gvB = jnp.logical_and(g >= g_loB, g < g_hiB)
            pb = pl.multiple_of(g * 16, 16)
            cvA = clbA[pl.ds(pb, 16)]
            cvB = clbB[pl.ds(pb, 16)]
            colb = jnp.full((16,), pb, dtype=jnp.int32)
            for p in range(16):
                ccA = jnp.where(gvA, cvA[p], jnp.int32(-1))
                ccB = jnp.where(gvB, cvB[p], jnp.int32(-1))
                fA0 = plsc.load_gather(htbA, [iota, colb + p])
                fA1 = plsc.load_gather(htbA, [rows1, colb + p])
                fB0 = plsc.load_gather(htbB, [iota, colb + p])
                fB1 = plsc.load_gather(htbB, [rows1, colb + p])
                sameA = ccA == prevA
                sameB = ccB == prevB
                a0A = jnp.maximum(fA0, jnp.where(sameA, a0A, 0.0))
                a1A = jnp.maximum(fA1, jnp.where(sameA, a1A, 0.0))
                a0B = jnp.maximum(fB0, jnp.where(sameB, a0B, 0.0))
                a1B = jnp.maximum(fB1, jnp.where(sameB, a1B, 0.0))
                irA = jnp.logical_and(ccA >= c_loA, ccA < c_loA + CPV)
                irB = jnp.logical_and(ccB >= c_loB, ccB < c_loB + CPV)
                rowA = jnp.where(irA, ccA - c_loW, RPAD - 1) * 32
                rowB = jnp.where(irB, ccB - c_loW, RPAD - 1) * 32
                poolb[pl.ds(rowA, 16)] = a0A
                poolb[pl.ds(rowA + 16, 16)] = a1A
                poolb[pl.ds(rowB, 16)] = a0B
                poolb[pl.ds(rowB + 16, 16)] = a1B
                prevA = ccA
                prevB = ccB
            return (prevA, a0A, a1A, prevB, a0B, a1B)

        return lax.fori_loop(0, GPT, grp, carry)

    lax.fori_loop(0, nt, tile_body,
                  (jnp.int32(-1), zf, zf, jnp.int32(-1), zf, zf))

    def gb_body(gb, carry):
        base = pl.multiple_of(gb * 512, 512)
        bidx = iota32 + base
        bg1v0 = wb[pl.ds(BG1O, 16)]
        bg1v1 = wb[pl.ds(BG1O + 16, 16)]
        bg2v0 = wb[pl.ds(BG2O, 16)]
        bg2v1 = wb[pl.ds(BG2O + 16, 16)]

        def l1(f, accs):
            pf = plsc.load_gather(poolb, [bidx + f])
            woff = pl.multiple_of(WG1O + f * 32, 16)
            wv0 = wb[pl.ds(woff, 16)]
            wv1 = wb[pl.ds(woff + 16, 16)]
            return (tuple(accs[j] + pf * wv0[j] for j in range(16))
                    + tuple(accs[16 + j] + pf * wv1[j] for j in range(16)))

        accs = lax.fori_loop(
            0, 32, l1,
            tuple(_splat(bg1v0[j]) for j in range(16))
            + tuple(_splat(bg1v1[j]) for j in range(16)))
        for j in range(32):
            g1b[pl.ds(j * 16, 16)] = jnp.maximum(accs[j], 0.0)

        def l2(f, accs):
            gf = g1b[pl.ds(pl.multiple_of(f * 16, 16), 16)]
            woff = pl.multiple_of(WG2O + f * 32, 16)
            wv0 = wb[pl.ds(woff, 16)]
            wv1 = wb[pl.ds(woff + 16, 16)]
            return (tuple(accs[j] + gf * wv0[j] for j in range(16))
                    + tuple(accs[16 + j] + gf * wv1[j] for j in range(16)))

        accs = lax.fori_loop(
            0, 32, l2,
            tuple(_splat(bg2v0[j]) for j in range(16))
            + tuple(_splat(bg2v1[j]) for j in range(16)))
        for j in range(32):
            plsc.store_scatter(poolb, [bidx + j], jnp.maximum(accs[j], 0.0))
        return carry

    lax.fori_loop(0, RPAD // 16, gb_body, 0)

    out_off = pl.multiple_of(wid * (CPW * 32), 128)
    pltpu.sync_copy(poolb.at[pl.ds(0, CPW * 32)],
                    out_h.at[pl.ds(out_off, CPW * 32)])


_pool = functools.partial(
    pl.kernel,
    out_type=jax.ShapeDtypeStruct((OUTR * 32,), jnp.float32),
    mesh=plsc.VectorSubcoreMesh(core_axis_name="c", subcore_axis_name="s"),
    compiler_params=pltpu.CompilerParams(needs_layout_passes=False),
    scratch_types=[
        pltpu.VMEM((32, TP), jnp.float32),
        pltpu.VMEM((32, TP), jnp.float32),
        pltpu.VMEM((TP,), jnp.int32),
        pltpu.VMEM((TP,), jnp.int32),
        pltpu.VMEM((WTOT,), jnp.float32),
        pltpu.VMEM((80,), jnp.int32),
        pltpu.VMEM((512,), jnp.float32),
        pltpu.VMEM((RPAD * 32,), jnp.float32),
        pltpu.SMEM((80,), jnp.int32),
    ],
)(_pool_body)


def kernel(points, cluster, W1, b1, W2, b2, Wg1, bg1, Wg2, bg2):
    cl = cluster.astype(jnp.int32)
    ht = _point_mlp(points.T, W1.T, b1.reshape(16, 1),
                    W2.T, b2.reshape(32, 1))
    w = jnp.concatenate([
        Wg1.reshape(-1), bg1, Wg2.reshape(-1), bg2,
    ]).astype(jnp.float32)
    splits = jnp.arange(0, NS + 1, dtype=jnp.int32) * CPV
    bd = jnp.searchsorted(cl, splits).astype(jnp.int32)
    bd = jnp.concatenate([bd, jnp.zeros((15,), jnp.int32)])
    out = _pool(ht, cl, w, bd)
    return out.reshape(OUTR, 32)[:C]


# speedup vs baseline: 1.1398x; 1.1398x over previous
import functools

import jax
import jax.numpy as jnp
from jax import lax
from jax.experimental import pallas as pl
from jax.experimental.pallas import tpu as pltpu
from jax.experimental.pallas import tpu_sc as plsc

N = 1600000
C = 50000
NW = 32
CPW = 1564
RPAD = 1568
OUTR = NW * CPW
TP = 1280

BLKN = 6400

WG1O, BG1O, WG2O, BG2O = 0, 1024, 1056, 2080
WTOT = 2112


def _mlp_body(p_ref, w1_ref, b1_ref, w2_ref, b2_ref, h_ref):
    p = p_ref[...]
    h1 = jnp.maximum(
        jnp.dot(w1_ref[...], p, preferred_element_type=jnp.float32)
        + b1_ref[...], 0.0)
    h2 = jnp.maximum(
        jnp.dot(w2_ref[...], h1, preferred_element_type=jnp.float32)
        + b2_ref[...], 0.0)
    h_ref[...] = h2


_point_mlp = pl.pallas_call(
    _mlp_body,
    grid=(N // BLKN,),
    in_specs=[
        pl.BlockSpec((3, BLKN), lambda i: (0, i)),
        pl.BlockSpec((16, 3), lambda i: (0, 0)),
        pl.BlockSpec((16, 1), lambda i: (0, 0)),
        pl.BlockSpec((32, 16), lambda i: (0, 0)),
        pl.BlockSpec((32, 1), lambda i: (0, 0)),
    ],
    out_specs=pl.BlockSpec((32, BLKN), lambda i: (0, i)),
    out_shape=jax.ShapeDtypeStruct((32, N), jnp.float32),
)


def _splat(s):
    return jnp.full((16,), s, dtype=jnp.float32)


def _pool_body(ht_h, cl_h, w_h, bd_h, out_h,
               htb, clb, wb, bdb, g1b, poolb, bsm):
    wid = lax.axis_index("s") * 2 + lax.axis_index("c")
    pltpu.sync_copy(w_h, wb)
    pltpu.sync_copy(bd_h, bdb)
    iota = lax.iota(jnp.int32, 16)
    iota32 = iota * 32
    rows1 = iota + 16
    c_lo = wid * CPW

    for i in range(3):
        v = bdb[pl.ds(i * 16, 16)]
        for l in range(16):
            bsm[i * 16 + l] = v[l]

    zf = jnp.zeros((16,), jnp.float32)

    def zbody(i, carry):
        poolb[pl.ds(pl.multiple_of(i * 16, 16), 16)] = zf
        return carry

    lax.fori_loop(0, RPAD * 32 // 16, zbody, 0)

    start = bsm[wid]
    end = bsm[wid + 1]
    t0 = start // TP
    t1 = (end + TP - 1) // TP

    def tile_body(t, carry):
        tb = t * TP
        off = pl.multiple_of(tb, TP)
        pltpu.sync_copy(ht_h.at[:, pl.ds(off, TP)], htb)
        pltpu.sync_copy(cl_h.at[pl.ds(off, TP)], clb)
        s_t = jnp.maximum(start, tb)
        e_t = jnp.minimum(end, tb + TP)
        g_lo = (s_t - tb) // 16
        g_hi = (e_t - tb + 15) // 16

        def grp(g, carry):
            prev0, b0, b1 = carry
            pb = pl.multiple_of(g * 16, 16)
            cv = clb[pl.ds(pb, 16)]
            colb = jnp.full((16,), pb, dtype=jnp.int32)
            c_first = cv[0]
            uniform = c_first == cv[15]

            def fast(carry):
                prev, a0, a1 = carry
                m0 = [jnp.maximum(
                    plsc.load_gather(htb, [iota, colb + i]),
                    plsc.load_gather(htb, [iota, colb + (i + 8)]))
                    for i in range(8)]
                m1 = [jnp.maximum(
                    plsc.load_gather(htb, [rows1, colb + i]),
                    plsc.load_gather(htb, [rows1, colb + (i + 8)]))
                    for i in range(8)]
                for lvl in (4, 2, 1):
                    m0 = [jnp.maximum(m0[i], m0[i + lvl]) for i in range(lvl)]
                    m1 = [jnp.maximum(m1[i], m1[i + lvl]) for i in range(lvl)]
                same = c_first == prev
                a0 = jnp.maximum(m0[0], jnp.where(same, a0, 0.0))
                a1 = jnp.maximum(m1[0], jnp.where(same, a1, 0.0))
                in_r = jnp.logical_and(c_first >= c_lo, c_first < c_lo + CPW)
                row = jnp.where(in_r, c_first - c_lo, RPAD - 1) * 32
                poolb[pl.ds(row, 16)] = a0
                poolb[pl.ds(row + 16, 16)] = a1
                return (c_first, a0, a1)

            def slow(carry):
                prev, a0, a1 = carry
                for p in range(16):
                    cc = cv[p]
                    f0 = plsc.load_gather(htb, [iota, colb + p])
                    f1 = plsc.load_gather(htb, [rows1, colb + p])
                    same = cc == prev
                    a0 = jnp.maximum(f0, jnp.where(same, a0, 0.0))
                    a1 = jnp.maximum(f1, jnp.where(same, a1, 0.0))
                    in_r = jnp.logical_and(cc >= c_lo, cc < c_lo + CPW)
                    row = jnp.where(in_r, cc - c_lo, RPAD - 1) * 32
                    poolb[pl.ds(row, 16)] = a0
                    poolb[pl.ds(row + 16, 16)] = a1
                    prev = cc
                return (prev, a0, a1)

            return lax.cond(uniform, fast, slow, (prev0, b0, b1))

        return lax.fori_loop(g_lo, g_hi, grp, carry)

    lax.fori_loop(t0, t1, tile_body, (jnp.int32(-1), zf, zf))

    def gb_body(gb, carry):
        base = pl.multiple_of(gb * 512, 512)
        bidx = iota32 + base
        bg1v0 = wb[pl.ds(BG1O, 16)]
        bg1v1 = wb[pl.ds(BG1O + 16, 16)]
        bg2v0 = wb[pl.ds(BG2O, 16)]
        bg2v1 = wb[pl.ds(BG2O + 16, 16)]

        def l1(f, accs):
            pf = plsc.load_gather(poolb, [bidx + f])
            woff = pl.multiple_of(WG1O + f * 32, 16)
            wv0 = wb[pl.ds(woff, 16)]
            wv1 = wb[pl.ds(woff + 16, 16)]
            return (tuple(accs[j] + pf * wv0[j] for j in range(16))
                    + tuple(accs[16 + j] + pf * wv1[j] for j in range(16)))

        accs = lax.fori_loop(
            0, 32, l1,
            tuple(_splat(bg1v0[j]) for j in range(16))
            + tuple(_splat(bg1v1[j]) for j in range(16)))
        for j in range(32):
            g1b[pl.ds(j * 16, 16)] = jnp.maximum(accs[j], 0.0)

        def l2(f, accs):
            gf = g1b[pl.ds(pl.multiple_of(f * 16, 16), 16)]
            woff = pl.multiple_of(WG2O + f * 32, 16)
            wv0 = wb[pl.ds(woff, 16)]
            wv1 = wb[pl.ds(woff + 16, 16)]
            return (tuple(accs[j] + gf * wv0[j] for j in range(16))
                    + tuple(accs[16 + j] + gf * wv1[j] for j in range(16)))

        accs = lax.fori_loop(
            0, 32, l2,
            tuple(_splat(bg2v0[j]) for j in range(16))
            + tuple(_splat(bg2v1[j]) for j in range(16)))
        for j in range(32):
            plsc.store_scatter(poolb, [bidx + j], jnp.maximum(accs[j], 0.0))
        return carry

    lax.fori_loop(0, RPAD // 16, gb_body, 0)

    out_off = pl.multiple_of(wid * (CPW * 32), 32)
    pltpu.sync_copy(poolb.at[pl.ds(0, CPW * 32)],
                    out_h.at[pl.ds(out_off, CPW * 32)])


_pool = functools.partial(
    pl.kernel,
    out_type=jax.ShapeDtypeStruct((OUTR * 32,), jnp.float32),
    mesh=plsc.VectorSubcoreMesh(core_axis_name="c", subcore_axis_name="s"),
    compiler_params=pltpu.CompilerParams(needs_layout_passes=False),
    scratch_types=[
        pltpu.VMEM((32, TP), jnp.float32),
        pltpu.VMEM((TP,), jnp.int32),
        pltpu.VMEM((WTOT,), jnp.float32),
        pltpu.VMEM((48,), jnp.int32),
        pltpu.VMEM((512,), jnp.float32),
        pltpu.VMEM((RPAD * 32,), jnp.float32),
        pltpu.SMEM((48,), jnp.int32),
    ],
)(_pool_body)


def kernel(points, cluster, W1, b1, W2, b2, Wg1, bg1, Wg2, bg2):
    cl = cluster.astype(jnp.int32)
    ht = _point_mlp(points.T, W1.T, b1.reshape(16, 1),
                    W2.T, b2.reshape(32, 1))
    w = jnp.concatenate([
        Wg1.reshape(-1), bg1, Wg2.reshape(-1), bg2,
    ]).astype(jnp.float32)
    splits = jnp.arange(0, NW + 1, dtype=jnp.int32) * CPW
    bd = jnp.searchsorted(cl, splits).astype(jnp.int32)
    bd = jnp.concatenate([bd, jnp.zeros((15,), jnp.int32)])
    out = _pool(ht, cl, w, bd)
    return out.reshape(OUTR, 32)[:C]


# speedup vs baseline: 1.1502x; 1.0091x over previous
import functools

import jax
import jax.numpy as jnp
from jax import lax
from jax.experimental import pallas as pl
from jax.experimental.pallas import tpu as pltpu
from jax.experimental.pallas import tpu_sc as plsc

N = 1600000
C = 50000
NW = 32
CPW = 1564
RPAD = 1568
OUTR = NW * CPW
TP = 1280

BLKN = 6400

WG1O, BG1O, WG2O, BG2O = 0, 1024, 1056, 2080
WTOT = 2112


def _mlp_body(p_ref, w1_ref, b1_ref, w2_ref, b2_ref, h_ref):
    p = p_ref[...]
    h1 = jnp.maximum(
        jnp.dot(w1_ref[...], p, preferred_element_type=jnp.float32)
        + b1_ref[...], 0.0)
    h2 = jnp.maximum(
        jnp.dot(w2_ref[...], h1, preferred_element_type=jnp.float32)
        + b2_ref[...], 0.0)
    h_ref[...] = h2.T


_point_mlp = pl.pallas_call(
    _mlp_body,
    grid=(N // BLKN,),
    in_specs=[
        pl.BlockSpec((3, BLKN), lambda i: (0, i)),
        pl.BlockSpec((16, 3), lambda i: (0, 0)),
        pl.BlockSpec((16, 1), lambda i: (0, 0)),
        pl.BlockSpec((32, 16), lambda i: (0, 0)),
        pl.BlockSpec((32, 1), lambda i: (0, 0)),
    ],
    out_specs=pl.BlockSpec((BLKN, 32), lambda i: (i, 0)),
    out_shape=jax.ShapeDtypeStruct((N, 32), jnp.float32),
)


def _splat(s):
    return jnp.full((16,), s, dtype=jnp.float32)


def _pool_body(ht_h, cl_h, w_h, bd_h, out_h,
               htb, clb, wb, bdb, g1b, poolb, bsm):
    wid = lax.axis_index("s") * 2 + lax.axis_index("c")
    pltpu.sync_copy(w_h, wb)
    pltpu.sync_copy(bd_h, bdb)
    iota = lax.iota(jnp.int32, 16)
    iota32 = iota * 32
    rows1 = iota + 16
    c_lo = wid * CPW

    for i in range(3):
        v = bdb[pl.ds(i * 16, 16)]
        for l in range(16):
            bsm[i * 16 + l] = v[l]

    zf = jnp.zeros((16,), jnp.float32)

    def zbody(i, carry):
        poolb[pl.ds(pl.multiple_of(i * 16, 16), 16)] = zf
        return carry

    lax.fori_loop(0, RPAD * 32 // 16, zbody, 0)

    start = bsm[wid]
    end = bsm[wid + 1]
    t0 = start // TP
    t1 = (end + TP - 1) // TP

    def tile_body(t, carry):
        tb = t * TP
        off = pl.multiple_of(tb, TP)
        pltpu.sync_copy(ht_h.at[pl.ds(off * 32, TP * 32)], htb)
        pltpu.sync_copy(cl_h.at[pl.ds(off, TP)], clb)
        s_t = jnp.maximum(start, tb)
        e_t = jnp.minimum(end, tb + TP)
        g_lo = (s_t - tb) // 16
        g_hi = (e_t - tb + 15) // 16

        def grp(g, carry):
            prev0, b0, b1 = carry
            pb = pl.multiple_of(g * 16, 16)
            cv = clb[pl.ds(pb, 16)]
            pb32 = pl.multiple_of(g * 512, 512)
            c_first = cv[0]
            uniform = c_first == cv[15]

            def fast(carry):
                prev, a0, a1 = carry
                m0 = [jnp.maximum(
                    htb[pl.ds(pb32 + i * 32, 16)],
                    htb[pl.ds(pb32 + (i + 8) * 32, 16)])
                    for i in range(8)]
                m1 = [jnp.maximum(
                    htb[pl.ds(pb32 + i * 32 + 16, 16)],
                    htb[pl.ds(pb32 + (i + 8) * 32 + 16, 16)])
                    for i in range(8)]
                for lvl in (4, 2, 1):
                    m0 = [jnp.maximum(m0[i], m0[i + lvl]) for i in range(lvl)]
                    m1 = [jnp.maximum(m1[i], m1[i + lvl]) for i in range(lvl)]
                same = c_first == prev
                a0 = jnp.maximum(m0[0], jnp.where(same, a0, 0.0))
                a1 = jnp.maximum(m1[0], jnp.where(same, a1, 0.0))
                in_r = jnp.logical_and(c_first >= c_lo, c_first < c_lo + CPW)
                row = jnp.where(in_r, c_first - c_lo, RPAD - 1) * 32
                poolb[pl.ds(row, 16)] = a0
                poolb[pl.ds(row + 16, 16)] = a1
                return (c_first, a0, a1)

            def slow(carry):
                prev, a0, a1 = carry
                for p in range(16):
                    cc = cv[p]
                    f0 = htb[pl.ds(pb32 + p * 32, 16)]
                    f1 = htb[pl.ds(pb32 + p * 32 + 16, 16)]
                    same = cc == prev
                    a0 = jnp.maximum(f0, jnp.where(same, a0, 0.0))
                    a1 = jnp.maximum(f1, jnp.where(same, a1, 0.0))
                    in_r = jnp.logical_and(cc >= c_lo, cc < c_lo + CPW)
                    row = jnp.where(in_r, cc - c_lo, RPAD - 1) * 32
                    poolb[pl.ds(row, 16)] = a0
                    poolb[pl.ds(row + 16, 16)] = a1
                    prev = cc
                return (prev, a0, a1)

            return lax.cond(uniform, fast, slow, (prev0, b0, b1))

        return lax.fori_loop(g_lo, g_hi, grp, carry)

    lax.fori_loop(t0, t1, tile_body, (jnp.int32(-1), zf, zf))

    def gb_body(gb, carry):
        base = pl.multiple_of(gb * 512, 512)
        bidx = iota32 + base
        bg1v0 = wb[pl.ds(BG1O, 16)]
        bg1v1 = wb[pl.ds(BG1O + 16, 16)]
        bg2v0 = wb[pl.ds(BG2O, 16)]
        bg2v1 = wb[pl.ds(BG2O + 16, 16)]

        def l1(f, accs):
            pf = plsc.load_gather(poolb, [bidx + f])
            woff = pl.multiple_of(WG1O + f * 32, 16)
            wv0 = wb[pl.ds(woff, 16)]
            wv1 = wb[pl.ds(woff + 16, 16)]
            return (tuple(accs[j] + pf * wv0[j] for j in range(16))
                    + tuple(accs[16 + j] + pf * wv1[j] for j in range(16)))

        accs = lax.fori_loop(
            0, 32, l1,
            tuple(_splat(bg1v0[j]) for j in range(16))
            + tuple(_splat(bg1v1[j]) for j in range(16)))
        for j in range(32):
            g1b[pl.ds(j * 16, 16)] = jnp.maximum(accs[j], 0.0)

        def l2(f, accs):
            gf = g1b[pl.ds(pl.multiple_of(f * 16, 16), 16)]
            woff = pl.multiple_of(WG2O + f * 32, 16)
            wv0 = wb[pl.ds(woff, 16)]
            wv1 = wb[pl.ds(woff + 16, 16)]
            return (tuple(accs[j] + gf * wv0[j] for j in range(16))
                    + tuple(accs[16 + j] + gf * wv1[j] for j in range(16)))

        accs = lax.fori_loop(
            0, 32, l2,
            tuple(_splat(bg2v0[j]) for j in range(16))
            + tuple(_splat(bg2v1[j]) for j in range(16)))
        for j in range(32):
            plsc.store_scatter(poolb, [bidx + j], jnp.maximum(accs[j], 0.0))
        return carry

    lax.fori_loop(0, RPAD // 16, gb_body, 0)

    out_off = pl.multiple_of(wid * (CPW * 32), 32)
    pltpu.sync_copy(poolb.at[pl.ds(0, CPW * 32)],
                    out_h.at[pl.ds(out_off, CPW * 32)])


_pool = functools.partial(
    pl.kernel,
    out_type=jax.ShapeDtypeStruct((OUTR * 32,), jnp.float32),
    mesh=plsc.VectorSubcoreMesh(core_axis_name="c", subcore_axis_name="s"),
    compiler_params=pltpu.CompilerParams(needs_layout_passes=False),
    scratch_types=[
        pltpu.VMEM((TP * 32,), jnp.float32),
        pltpu.VMEM((TP,), jnp.int32),
        pltpu.VMEM((WTOT,), jnp.float32),
        pltpu.VMEM((48,), jnp.int32),
        pltpu.VMEM((512,), jnp.float32),
        pltpu.VMEM((RPAD * 32,), jnp.float32),
        pltpu.SMEM((48,), jnp.int32),
    ],
)(_pool_body)


def kernel(points, cluster, W1, b1, W2, b2, Wg1, bg1, Wg2, bg2):
    cl = cluster.astype(jnp.int32)
    ht = _point_mlp(points.T, W1.T, b1.reshape(16, 1),
                    W2.T, b2.reshape(32, 1)).reshape(-1)
    w = jnp.concatenate([
        Wg1.reshape(-1), bg1, Wg2.reshape(-1), bg2,
    ]).astype(jnp.float32)
    splits = jnp.arange(0, NW + 1, dtype=jnp.int32) * CPW
    bd = jnp.searchsorted(cl, splits).astype(jnp.int32)
    bd = jnp.concatenate([bd, jnp.zeros((15,), jnp.int32)])
    out = _pool(ht, cl, w, bd)
    return out.reshape(OUTR, 32)[:C]
